# Initial kernel scaffold; baseline (speedup 1.0000x reference)
import jax
import jax.numpy as jnp
from jax.experimental import pallas as pl


def kernel(emg_x, eeg_x, emg_edge_index, eeg_edge_index, emg_params, eeg_params):
    raise NotImplementedError("write your pallas kernel here")


# speedup vs baseline: 5.5566x; 5.5566x over previous
import functools

import jax
import jax.numpy as jnp
from jax import lax
from jax.experimental import pallas as pl
from jax.experimental.pallas import tpu as pltpu
from jax.experimental.pallas import tpu_sc as plsc

_TILES = 16
_CORES = 2
_CHUNK = 80


@functools.lru_cache(maxsize=None)
def _make_segment_sum2(n, e, dh):
    ept = e // _TILES
    nch = ept // _CHUNK
    rpt = (n // _TILES) // 8 * 8
    tail = _TILES * rpt
    rem = n - tail
    mesh = plsc.VectorSubcoreMesh(
        core_axis_name="c", subcore_axis_name="s",
        num_cores=_CORES, num_subcores=_TILES)

    @functools.partial(
        pl.kernel,
        out_type=[jax.ShapeDtypeStruct((2, n, dh), jnp.float32),
                  jax.ShapeDtypeStruct((2, n, dh), jnp.float32)],
        mesh=mesh,
        compiler_params=pltpu.CompilerParams(use_tc_tiling_on_sc=False),
        scratch_types=[
            pltpu.VMEM((nch, _CHUNK), jnp.int32),
            pltpu.VMEM((nch, _CHUNK), jnp.int32),
            pltpu.VMEM((_CHUNK, dh), jnp.float32),
            pltpu.VMEM_SHARED((n, dh), jnp.float32),
            pltpu.SemaphoreType.DMA,
        ],
    )
    def seg2(x0_hbm, x1_hbm, src0_hbm, dst0_hbm, src1_hbm, dst1_hbm, zrows_hbm,
             out0_hbm, out1_hbm, sidx, didx, rows, acc, sem):
        c = lax.axis_index("c")
        s = lax.axis_index("s")
        row_slice = pl.ds(s * rpt, rpt)
        tail_slice = pl.ds(tail, max(rem, 1))

        def stage_idx(src_hbm, dst_hbm):
            pltpu.sync_copy(src_hbm.at[s], sidx)
            pltpu.sync_copy(dst_hbm.at[s], didx)

        @pl.when(c == 0)
        def _():
            stage_idx(src0_hbm, dst0_hbm)

        @pl.when(c == 1)
        def _():
            stage_idx(src1_hbm, dst1_hbm)

        def zero_acc():
            pltpu.sync_copy(zrows_hbm.at[pl.ds(0, rpt)], acc.at[row_slice])
            if rem:
                @pl.when(s == _TILES - 1)
                def _():
                    pltpu.sync_copy(zrows_hbm.at[pl.ds(0, rem)],
                                    acc.at[tail_slice])

        def accumulate(x_hbm, phase):
            def body(j, carry):
                pltpu.async_copy(x_hbm.at[phase].at[sidx.at[j]], rows,
                                 sem).wait()
                pltpu.sync_copy(rows, acc.at[didx.at[j]], add=True)
                return carry

            lax.fori_loop(0, nch, body, 0)

        def writeout(out_hbm, phase):
            pltpu.sync_copy(acc.at[row_slice], out_hbm.at[phase].at[row_slice])
            if rem:
                @pl.when(s == _TILES - 1)
                def _():
                    pltpu.sync_copy(acc.at[tail_slice],
                                    out_hbm.at[phase].at[tail_slice])

        for phase in (0, 1):
            zero_acc()
            plsc.subcore_barrier()

            @pl.when(c == 0)
            def _():
                accumulate(x0_hbm, phase)

            @pl.when(c == 1)
            def _():
                accumulate(x1_hbm, phase)

            plsc.subcore_barrier()

            @pl.when(c == 0)
            def _():
                writeout(out0_hbm, phase)

            @pl.when(c == 1)
            def _():
                writeout(out1_hbm, phase)

            if phase == 0:
                plsc.subcore_barrier()

    return seg2


def _segment_sum2(x0h, x1h, idx0, idx1):
    _, n, dh = x0h.shape
    e = idx0.shape[1]
    shp = (_TILES, e // (_TILES * _CHUNK), _CHUNK)
    src0, dst0 = idx0[0].reshape(shp), idx0[1].reshape(shp)
    src1, dst1 = idx1[0].reshape(shp), idx1[1].reshape(shp)
    zrows = jnp.zeros(((n // _TILES) // 8 * 8, dh), jnp.float32)
    return _make_segment_sum2(n, e, dh)(
        x0h, x1h, src0, dst0, src1, dst1, zrows)


_BLK = 1000


def _mlp1_body(x_ref, agg_ref, w1a_ref, b1a_ref, w1b_ref, b1b_ref, w2a_ref,
               t_ref, th_ref):
    agg = jnp.concatenate([agg_ref[0], agg_ref[1]], axis=1)
    xa = x_ref[...] + agg
    g = jnp.maximum(
        jnp.dot(xa, w1a_ref[...], preferred_element_type=jnp.float32)
        + b1a_ref[...], 0.0)
    h = jnp.maximum(
        jnp.dot(g, w1b_ref[...], preferred_element_type=jnp.float32)
        + b1b_ref[...], 0.0)
    t = jnp.dot(h, w2a_ref[...], preferred_element_type=jnp.float32)
    t_ref[...] = t
    dh = t.shape[1] // 2
    th_ref[0] = t[:, :dh]
    th_ref[1] = t[:, dh:]


def _mlp1(x, agg_halves, p):
    n, d_in = x.shape
    hid = p["W1a"].shape[1]
    lat = p["W2a"].shape[1]
    grid = (n // _BLK,)
    full = lambda shape: pl.BlockSpec(shape, lambda i: (0,) * len(shape))
    return pl.pallas_call(
        _mlp1_body,
        grid=grid,
        in_specs=[
            pl.BlockSpec((_BLK, d_in), lambda i: (i, 0)),
            pl.BlockSpec((2, _BLK, d_in // 2), lambda i: (0, i, 0)),
            full((d_in, hid)), full((1, hid)),
            full((hid, hid)), full((1, hid)),
            full((hid, lat)),
        ],
        out_specs=[
            pl.BlockSpec((_BLK, lat), lambda i: (i, 0)),
            pl.BlockSpec((2, _BLK, lat // 2), lambda i: (0, i, 0)),
        ],
        out_shape=[
            jax.ShapeDtypeStruct((n, lat), jnp.float32),
            jax.ShapeDtypeStruct((2, n, lat // 2), jnp.float32),
        ],
    )(x, agg_halves, p["W1a"], p["b1a"].reshape(1, -1), p["W1b"],
      p["b1b"].reshape(1, -1), p["W2a"])


def _mlp2_body(t_ref, aggt_ref, b2a_ref, w2b_ref, b2b_ref, wp_ref, bp_ref,
               o_ref):
    aggt = jnp.concatenate([aggt_ref[0], aggt_ref[1]], axis=1)
    z = jnp.maximum(t_ref[...] + aggt + b2a_ref[...], 0.0)
    h2 = jnp.dot(z, w2b_ref[...], preferred_element_type=jnp.float32) \
        + b2b_ref[...]
    o_ref[...] = jnp.dot(h2, wp_ref[...], preferred_element_type=jnp.float32) \
        + bp_ref[...]


def _mlp2(t, aggt_halves, p):
    n, lat = t.shape
    grid = (n // _BLK,)
    full = lambda shape: pl.BlockSpec(shape, lambda i: (0,) * len(shape))
    return pl.pallas_call(
        _mlp2_body,
        grid=grid,
        in_specs=[
            pl.BlockSpec((_BLK, lat), lambda i: (i, 0)),
            pl.BlockSpec((2, _BLK, lat // 2), lambda i: (0, i, 0)),
            full((1, lat)),
            full((lat, lat)), full((1, lat)),
            full((lat, lat)), full((1, lat)),
        ],
        out_specs=pl.BlockSpec((_BLK, lat), lambda i: (i, 0)),
        out_shape=jax.ShapeDtypeStruct((n, lat), jnp.float32),
    )(t, aggt_halves, p["b2a"].reshape(1, -1), p["W2b"], p["b2b"].reshape(1, -1),
      p["Wp"], p["bp"].reshape(1, -1))


def _split_cols(x):
    n, d = x.shape
    return jnp.stack([x[:, :d // 2], x[:, d // 2:]])


def kernel(emg_x, eeg_x, emg_edge_index, eeg_edge_index, emg_params,
           eeg_params):
    agg_emg, agg_eeg = _segment_sum2(
        _split_cols(emg_x), _split_cols(eeg_x), emg_edge_index, eeg_edge_index)
    t_emg, th_emg = _mlp1(emg_x, agg_emg, emg_params)
    t_eeg, th_eeg = _mlp1(eeg_x, agg_eeg, eeg_params)
    aggt_emg, aggt_eeg = _segment_sum2(
        th_emg, th_eeg, emg_edge_index, eeg_edge_index)
    o_emg = _mlp2(t_emg, aggt_emg, emg_params)
    o_eeg = _mlp2(t_eeg, aggt_eeg, eeg_params)
    return jnp.concatenate([o_emg, o_eeg], axis=0)


# speedup vs baseline: 11.4821x; 2.0664x over previous
import functools

import jax
import jax.numpy as jnp
from jax import lax
from jax.experimental import pallas as pl
from jax.experimental.pallas import tpu as pltpu
from jax.experimental.pallas import tpu_sc as plsc

_TILES = 16
_CORES = 2
_CHUNK = 80
_NBUF = 5
_LOOK = 3


@functools.lru_cache(maxsize=None)
def _make_segment_sum2(n, e, dh):
    ept = e // _TILES
    nch = ept // _CHUNK
    rpt = (n // _TILES) // 8 * 8
    tail = _TILES * rpt
    rem = n - tail
    mesh = plsc.VectorSubcoreMesh(
        core_axis_name="c", subcore_axis_name="s",
        num_cores=_CORES, num_subcores=_TILES)

    @functools.partial(
        pl.kernel,
        out_type=[jax.ShapeDtypeStruct((2, n, dh), jnp.float32),
                  jax.ShapeDtypeStruct((2, n, dh), jnp.float32)],
        mesh=mesh,
        compiler_params=pltpu.CompilerParams(use_tc_tiling_on_sc=False),
        scratch_types=[
            pltpu.VMEM((nch, _CHUNK), jnp.int32),
            pltpu.VMEM((nch, _CHUNK), jnp.int32),
            pltpu.VMEM((_NBUF, _CHUNK, dh), jnp.float32),
            pltpu.VMEM_SHARED((n, dh), jnp.float32),
            pltpu.SemaphoreType.DMA,
        ],
    )
    def seg2(x0_hbm, x1_hbm, src0_hbm, dst0_hbm, src1_hbm, dst1_hbm, zrows_hbm,
             out0_hbm, out1_hbm, sidx, didx, rows, acc, gsem):
        c = lax.axis_index("c")
        s = lax.axis_index("s")
        row_slice = pl.ds(s * rpt, rpt)
        tail_slice = pl.ds(tail, max(rem, 1))

        def stage_idx(src_hbm, dst_hbm):
            pltpu.sync_copy(src_hbm.at[s], sidx)
            pltpu.sync_copy(dst_hbm.at[s], didx)

        @pl.when(c == 0)
        def _():
            stage_idx(src0_hbm, dst0_hbm)

        @pl.when(c == 1)
        def _():
            stage_idx(src1_hbm, dst1_hbm)

        def zero_acc():
            pltpu.sync_copy(zrows_hbm.at[pl.ds(0, rpt)], acc.at[row_slice])
            if rem:
                @pl.when(s == _TILES - 1)
                def _():
                    pltpu.sync_copy(zrows_hbm.at[pl.ds(0, rem)],
                                    acc.at[tail_slice])

        def accumulate(x_hbm, phase):
            def fire_gather(g, b):
                pltpu.async_copy(x_hbm.at[phase].at[sidx.at[g]], rows.at[b],
                                 gsem)

            def wait_gather(g, b):
                pltpu.make_async_copy(x_hbm.at[phase].at[sidx.at[g]],
                                      rows.at[b], gsem).wait()

            for g in range(_LOOK):
                fire_gather(g, g % _NBUF)

            def body(i, carry):
                for b in range(_NBUF):
                    g = i + b
                    wait_gather(g, b)
                    pltpu.sync_copy(rows.at[b], acc.at[didx.at[g]], add=True)

                    @pl.when(g + _LOOK < nch)
                    def _():
                        fire_gather(g + _LOOK, (b + _LOOK) % _NBUF)
                return carry

            lax.fori_loop(0, nch // _NBUF, lambda i, c: body(i * _NBUF, c), 0)

        def writeout(out_hbm, phase):
            pltpu.sync_copy(acc.at[row_slice], out_hbm.at[phase].at[row_slice])
            if rem:
                @pl.when(s == _TILES - 1)
                def _():
                    pltpu.sync_copy(acc.at[tail_slice],
                                    out_hbm.at[phase].at[tail_slice])

        for phase in (0, 1):
            zero_acc()
            plsc.subcore_barrier()

            @pl.when(c == 0)
            def _():
                accumulate(x0_hbm, phase)

            @pl.when(c == 1)
            def _():
                accumulate(x1_hbm, phase)

            plsc.subcore_barrier()

            @pl.when(c == 0)
            def _():
                writeout(out0_hbm, phase)

            @pl.when(c == 1)
            def _():
                writeout(out1_hbm, phase)

            if phase == 0:
                plsc.subcore_barrier()

    return seg2


def _segment_sum2(x0h, x1h, idx0, idx1):
    _, n, dh = x0h.shape
    e = idx0.shape[1]
    shp = (_TILES, e // (_TILES * _CHUNK), _CHUNK)
    src0, dst0 = idx0[0].reshape(shp), idx0[1].reshape(shp)
    src1, dst1 = idx1[0].reshape(shp), idx1[1].reshape(shp)
    zrows = jnp.zeros(((n // _TILES) // 8 * 8, dh), jnp.float32)
    return _make_segment_sum2(n, e, dh)(
        x0h, x1h, src0, dst0, src1, dst1, zrows)


_BLK = 1000


def _mlp1_body(x_ref, agg_ref, w1a_ref, b1a_ref, w1b_ref, b1b_ref, w2a_ref,
               t_ref, th_ref):
    agg = jnp.concatenate([agg_ref[0], agg_ref[1]], axis=1)
    xa = x_ref[...] + agg
    g = jnp.maximum(
        jnp.dot(xa, w1a_ref[...], preferred_element_type=jnp.float32)
        + b1a_ref[...], 0.0)
    h = jnp.maximum(
        jnp.dot(g, w1b_ref[...], preferred_element_type=jnp.float32)
        + b1b_ref[...], 0.0)
    t = jnp.dot(h, w2a_ref[...], preferred_element_type=jnp.float32)
    t_ref[...] = t
    dh = t.shape[1] // 2
    th_ref[0] = t[:, :dh]
    th_ref[1] = t[:, dh:]


def _mlp1(x, agg_halves, p):
    n, d_in = x.shape
    hid = p["W1a"].shape[1]
    lat = p["W2a"].shape[1]
    grid = (n // _BLK,)
    full = lambda shape: pl.BlockSpec(shape, lambda i: (0,) * len(shape))
    return pl.pallas_call(
        _mlp1_body,
        grid=grid,
        in_specs=[
            pl.BlockSpec((_BLK, d_in), lambda i: (i, 0)),
            pl.BlockSpec((2, _BLK, d_in // 2), lambda i: (0, i, 0)),
            full((d_in, hid)), full((1, hid)),
            full((hid, hid)), full((1, hid)),
            full((hid, lat)),
        ],
        out_specs=[
            pl.BlockSpec((_BLK, lat), lambda i: (i, 0)),
            pl.BlockSpec((2, _BLK, lat // 2), lambda i: (0, i, 0)),
        ],
        out_shape=[
            jax.ShapeDtypeStruct((n, lat), jnp.float32),
            jax.ShapeDtypeStruct((2, n, lat // 2), jnp.float32),
        ],
    )(x, agg_halves, p["W1a"], p["b1a"].reshape(1, -1), p["W1b"],
      p["b1b"].reshape(1, -1), p["W2a"])


def _mlp2_body(t_ref, aggt_ref, b2a_ref, w2b_ref, b2b_ref, wp_ref, bp_ref,
               o_ref):
    aggt = jnp.concatenate([aggt_ref[0], aggt_ref[1]], axis=1)
    z = jnp.maximum(t_ref[...] + aggt + b2a_ref[...], 0.0)
    h2 = jnp.dot(z, w2b_ref[...], preferred_element_type=jnp.float32) \
        + b2b_ref[...]
    o_ref[...] = jnp.dot(h2, wp_ref[...], preferred_element_type=jnp.float32) \
        + bp_ref[...]


def _mlp2(t, aggt_halves, p):
    n, lat = t.shape
    grid = (n // _BLK,)
    full = lambda shape: pl.BlockSpec(shape, lambda i: (0,) * len(shape))
    return pl.pallas_call(
        _mlp2_body,
        grid=grid,
        in_specs=[
            pl.BlockSpec((_BLK, lat), lambda i: (i, 0)),
            pl.BlockSpec((2, _BLK, lat // 2), lambda i: (0, i, 0)),
            full((1, lat)),
            full((lat, lat)), full((1, lat)),
            full((lat, lat)), full((1, lat)),
        ],
        out_specs=pl.BlockSpec((_BLK, lat), lambda i: (i, 0)),
        out_shape=jax.ShapeDtypeStruct((n, lat), jnp.float32),
    )(t, aggt_halves, p["b2a"].reshape(1, -1), p["W2b"], p["b2b"].reshape(1, -1),
      p["Wp"], p["bp"].reshape(1, -1))


def _split_cols(x):
    n, d = x.shape
    return jnp.stack([x[:, :d // 2], x[:, d // 2:]])


def kernel(emg_x, eeg_x, emg_edge_index, eeg_edge_index, emg_params,
           eeg_params):
    agg_emg, agg_eeg = _segment_sum2(
        _split_cols(emg_x), _split_cols(eeg_x), emg_edge_index, eeg_edge_index)
    t_emg, th_emg = _mlp1(emg_x, agg_emg, emg_params)
    t_eeg, th_eeg = _mlp1(eeg_x, agg_eeg, eeg_params)
    aggt_emg, aggt_eeg = _segment_sum2(
        th_emg, th_eeg, emg_edge_index, eeg_edge_index)
    o_emg = _mlp2(t_emg, aggt_emg, emg_params)
    o_eeg = _mlp2(t_eeg, aggt_eeg, eeg_params)
    return jnp.concatenate([o_emg, o_eeg], axis=0)


# speedup vs baseline: 12.4305x; 1.0826x over previous
import functools

import jax
import jax.numpy as jnp
from jax import lax
from jax.experimental import pallas as pl
from jax.experimental.pallas import tpu as pltpu
from jax.experimental.pallas import tpu_sc as plsc

_TILES = 16
_CORES = 2
_CHUNK = 80
_NBUF = 5
_LOOK = 3
_SCAT = 2


@functools.lru_cache(maxsize=None)
def _make_segment_sum2(n, e, dh):
    ept = e // _TILES
    nch = ept // _CHUNK
    rpt = (n // _TILES) // 8 * 8
    tail = _TILES * rpt
    rem = n - tail
    mesh = plsc.VectorSubcoreMesh(
        core_axis_name="c", subcore_axis_name="s",
        num_cores=_CORES, num_subcores=_TILES)

    @functools.partial(
        pl.kernel,
        out_type=[jax.ShapeDtypeStruct((2, n, dh), jnp.float32),
                  jax.ShapeDtypeStruct((2, n, dh), jnp.float32)],
        mesh=mesh,
        compiler_params=pltpu.CompilerParams(use_tc_tiling_on_sc=False),
        scratch_types=[
            pltpu.VMEM((nch, _CHUNK), jnp.int32),
            pltpu.VMEM((nch, _CHUNK), jnp.int32),
            pltpu.VMEM((_NBUF, _CHUNK, dh), jnp.float32),
            pltpu.VMEM_SHARED((n, dh), jnp.float32),
            pltpu.SemaphoreType.DMA,
            pltpu.SemaphoreType.DMA,
        ],
    )
    def seg2(x0_hbm, x1_hbm, src0_hbm, dst0_hbm, src1_hbm, dst1_hbm, zrows_hbm,
             out0_hbm, out1_hbm, sidx, didx, rows, acc, gsem, ssem):
        c = lax.axis_index("c")
        s = lax.axis_index("s")
        row_slice = pl.ds(s * rpt, rpt)
        tail_slice = pl.ds(tail, max(rem, 1))

        def stage_idx(src_hbm, dst_hbm):
            pltpu.sync_copy(src_hbm.at[s], sidx)
            pltpu.sync_copy(dst_hbm.at[s], didx)

        @pl.when(c == 0)
        def _():
            stage_idx(src0_hbm, dst0_hbm)

        @pl.when(c == 1)
        def _():
            stage_idx(src1_hbm, dst1_hbm)

        def zero_acc():
            pltpu.sync_copy(zrows_hbm.at[pl.ds(0, rpt)], acc.at[row_slice])
            if rem:
                @pl.when(s == _TILES - 1)
                def _():
                    pltpu.sync_copy(zrows_hbm.at[pl.ds(0, rem)],
                                    acc.at[tail_slice])

        def accumulate(x_hbm, phase):
            def fire_gather(g, b):
                pltpu.async_copy(x_hbm.at[phase].at[sidx.at[g]], rows.at[b],
                                 gsem)

            def wait_gather(g, b):
                pltpu.make_async_copy(x_hbm.at[phase].at[sidx.at[g]],
                                      rows.at[b], gsem).wait()

            def fire_scatter(g, b):
                pltpu.async_copy(rows.at[b], acc.at[didx.at[g]], ssem,
                                 add=True)

            def wait_scatter(g, b):
                pltpu.make_async_copy(rows.at[b], acc.at[didx.at[g]],
                                      ssem).wait()

            for g in range(_LOOK):
                fire_gather(g, g % _NBUF)

            def body(i, carry):
                for b in range(_NBUF):
                    g = i + b
                    wait_gather(g, b)
                    fire_scatter(g, b)

                    @pl.when(g + _LOOK < nch)
                    def _():
                        fire_gather(g + _LOOK, (b + _LOOK) % _NBUF)

                    @pl.when(g >= _SCAT)
                    def _():
                        wait_scatter(g - _SCAT, (b - _SCAT) % _NBUF)
                return carry

            lax.fori_loop(0, nch // _NBUF, lambda i, c: body(i * _NBUF, c), 0)
            for g in range(nch - _SCAT, nch):
                wait_scatter(g, g % _NBUF)

        def writeout(out_hbm, phase):
            pltpu.sync_copy(acc.at[row_slice], out_hbm.at[phase].at[row_slice])
            if rem:
                @pl.when(s == _TILES - 1)
                def _():
                    pltpu.sync_copy(acc.at[tail_slice],
                                    out_hbm.at[phase].at[tail_slice])

        for phase in (0, 1):
            zero_acc()
            plsc.subcore_barrier()

            @pl.when(c == 0)
            def _():
                accumulate(x0_hbm, phase)

            @pl.when(c == 1)
            def _():
                accumulate(x1_hbm, phase)

            plsc.subcore_barrier()

            @pl.when(c == 0)
            def _():
                writeout(out0_hbm, phase)

            @pl.when(c == 1)
            def _():
                writeout(out1_hbm, phase)

            if phase == 0:
                plsc.subcore_barrier()

    return seg2


def _segment_sum2(x0h, x1h, idx0, idx1):
    _, n, dh = x0h.shape
    e = idx0.shape[1]
    shp = (_TILES, e // (_TILES * _CHUNK), _CHUNK)
    src0, dst0 = idx0[0].reshape(shp), idx0[1].reshape(shp)
    src1, dst1 = idx1[0].reshape(shp), idx1[1].reshape(shp)
    zrows = jnp.zeros(((n // _TILES) // 8 * 8, dh), jnp.float32)
    return _make_segment_sum2(n, e, dh)(
        x0h, x1h, src0, dst0, src1, dst1, zrows)


_BLK = 1000


def _mlp1_body(x_ref, agg_ref, w1a_ref, b1a_ref, w1b_ref, b1b_ref, w2a_ref,
               t_ref, th_ref):
    agg = jnp.concatenate([agg_ref[0], agg_ref[1]], axis=1)
    xa = x_ref[...] + agg
    g = jnp.maximum(
        jnp.dot(xa, w1a_ref[...], preferred_element_type=jnp.float32)
        + b1a_ref[...], 0.0)
    h = jnp.maximum(
        jnp.dot(g, w1b_ref[...], preferred_element_type=jnp.float32)
        + b1b_ref[...], 0.0)
    t = jnp.dot(h, w2a_ref[...], preferred_element_type=jnp.float32)
    t_ref[...] = t
    dh = t.shape[1] // 2
    th_ref[0] = t[:, :dh]
    th_ref[1] = t[:, dh:]


def _mlp1(x, agg_halves, p):
    n, d_in = x.shape
    hid = p["W1a"].shape[1]
    lat = p["W2a"].shape[1]
    grid = (n // _BLK,)
    full = lambda shape: pl.BlockSpec(shape, lambda i: (0,) * len(shape))
    return pl.pallas_call(
        _mlp1_body,
        grid=grid,
        in_specs=[
            pl.BlockSpec((_BLK, d_in), lambda i: (i, 0)),
            pl.BlockSpec((2, _BLK, d_in // 2), lambda i: (0, i, 0)),
            full((d_in, hid)), full((1, hid)),
            full((hid, hid)), full((1, hid)),
            full((hid, lat)),
        ],
        out_specs=[
            pl.BlockSpec((_BLK, lat), lambda i: (i, 0)),
            pl.BlockSpec((2, _BLK, lat // 2), lambda i: (0, i, 0)),
        ],
        out_shape=[
            jax.ShapeDtypeStruct((n, lat), jnp.float32),
            jax.ShapeDtypeStruct((2, n, lat // 2), jnp.float32),
        ],
    )(x, agg_halves, p["W1a"], p["b1a"].reshape(1, -1), p["W1b"],
      p["b1b"].reshape(1, -1), p["W2a"])


def _mlp2_body(t_ref, aggt_ref, b2a_ref, w2b_ref, b2b_ref, wp_ref, bp_ref,
               o_ref):
    aggt = jnp.concatenate([aggt_ref[0], aggt_ref[1]], axis=1)
    z = jnp.maximum(t_ref[...] + aggt + b2a_ref[...], 0.0)
    h2 = jnp.dot(z, w2b_ref[...], preferred_element_type=jnp.float32) \
        + b2b_ref[...]
    o_ref[...] = jnp.dot(h2, wp_ref[...], preferred_element_type=jnp.float32) \
        + bp_ref[...]


def _mlp2(t, aggt_halves, p):
    n, lat = t.shape
    grid = (n // _BLK,)
    full = lambda shape: pl.BlockSpec(shape, lambda i: (0,) * len(shape))
    return pl.pallas_call(
        _mlp2_body,
        grid=grid,
        in_specs=[
            pl.BlockSpec((_BLK, lat), lambda i: (i, 0)),
            pl.BlockSpec((2, _BLK, lat // 2), lambda i: (0, i, 0)),
            full((1, lat)),
            full((lat, lat)), full((1, lat)),
            full((lat, lat)), full((1, lat)),
        ],
        out_specs=pl.BlockSpec((_BLK, lat), lambda i: (i, 0)),
        out_shape=jax.ShapeDtypeStruct((n, lat), jnp.float32),
    )(t, aggt_halves, p["b2a"].reshape(1, -1), p["W2b"], p["b2b"].reshape(1, -1),
      p["Wp"], p["bp"].reshape(1, -1))


def _split_cols(x):
    n, d = x.shape
    return jnp.stack([x[:, :d // 2], x[:, d // 2:]])


def kernel(emg_x, eeg_x, emg_edge_index, eeg_edge_index, emg_params,
           eeg_params):
    agg_emg, agg_eeg = _segment_sum2(
        _split_cols(emg_x), _split_cols(eeg_x), emg_edge_index, eeg_edge_index)
    t_emg, th_emg = _mlp1(emg_x, agg_emg, emg_params)
    t_eeg, th_eeg = _mlp1(eeg_x, agg_eeg, eeg_params)
    aggt_emg, aggt_eeg = _segment_sum2(
        th_emg, th_eeg, emg_edge_index, eeg_edge_index)
    o_emg = _mlp2(t_emg, aggt_emg, emg_params)
    o_eeg = _mlp2(t_eeg, aggt_eeg, eeg_params)
    return jnp.concatenate([o_emg, o_eeg], axis=0)


# speedup vs baseline: 13.1543x; 1.0582x over previous
import functools

import jax
import jax.numpy as jnp
from jax import lax
from jax.experimental import pallas as pl
from jax.experimental.pallas import tpu as pltpu
from jax.experimental.pallas import tpu_sc as plsc

_TILES = 16
_CORES = 2
_CHUNK = 80
_NBUF = 5
_LOOK = 3
_SCAT = 2


@functools.lru_cache(maxsize=None)
def _make_segment_sum2(n, e, dh):
    ept = e // _TILES
    nch = ept // _CHUNK
    rpt = (n // _TILES) // 8 * 8
    tail = _TILES * rpt
    rem = n - tail
    mesh = plsc.VectorSubcoreMesh(
        core_axis_name="c", subcore_axis_name="s",
        num_cores=_CORES, num_subcores=_TILES)

    @functools.partial(
        pl.kernel,
        out_type=[jax.ShapeDtypeStruct((2, n, dh), jnp.float32),
                  jax.ShapeDtypeStruct((2, n, dh), jnp.float32)],
        mesh=mesh,
        compiler_params=pltpu.CompilerParams(use_tc_tiling_on_sc=False),
        scratch_types=[
            pltpu.VMEM((nch, _CHUNK), jnp.int32),
            pltpu.VMEM((nch, _CHUNK), jnp.int32),
            pltpu.VMEM((_NBUF, _CHUNK, dh), jnp.float32),
            pltpu.VMEM_SHARED((n, dh), jnp.float32),
            pltpu.SemaphoreType.DMA,
            pltpu.SemaphoreType.DMA,
        ],
    )
    def seg2(x0_hbm, x1_hbm, src0_hbm, dst0_hbm, src1_hbm, dst1_hbm, zrows_hbm,
             out0_hbm, out1_hbm, sidx, didx, rows, acc, gsem, ssem):
        c = lax.axis_index("c")
        s = lax.axis_index("s")
        row_slice = pl.ds(s * rpt, rpt)
        tail_slice = pl.ds(tail, max(rem, 1))

        @pl.when(c == 0)
        def _():
            pltpu.sync_copy(dst0_hbm.at[s], didx)

        @pl.when(c == 1)
        def _():
            pltpu.sync_copy(dst1_hbm.at[s], didx)

        def zero_acc():
            pltpu.sync_copy(zrows_hbm.at[pl.ds(0, rpt)], acc.at[row_slice])
            if rem:
                @pl.when(s == _TILES - 1)
                def _():
                    pltpu.sync_copy(zrows_hbm.at[pl.ds(0, rem)],
                                    acc.at[tail_slice])

        def accumulate(x_hbm, src_hbm, phase):
            pltpu.sync_copy(src_hbm.at[phase, s], sidx)

            def fire_gather(g, b):
                pltpu.async_copy(x_hbm.at[sidx.at[g]], rows.at[b], gsem)

            def wait_gather(g, b):
                pltpu.make_async_copy(x_hbm.at[sidx.at[g]], rows.at[b],
                                      gsem).wait()

            def fire_scatter(g, b):
                pltpu.async_copy(rows.at[b], acc.at[didx.at[g]], ssem,
                                 add=True)

            def wait_scatter(g, b):
                pltpu.make_async_copy(rows.at[b], acc.at[didx.at[g]],
                                      ssem).wait()

            for g in range(_LOOK):
                fire_gather(g, g % _NBUF)

            def body(i, carry):
                for b in range(_NBUF):
                    g = i + b
                    wait_gather(g, b)
                    fire_scatter(g, b)

                    @pl.when(g + _LOOK < nch)
                    def _():
                        fire_gather(g + _LOOK, (b + _LOOK) % _NBUF)

                    @pl.when(g >= _SCAT)
                    def _():
                        wait_scatter(g - _SCAT, (b - _SCAT) % _NBUF)
                return carry

            lax.fori_loop(0, nch // _NBUF, lambda i, c: body(i * _NBUF, c), 0)
            for g in range(nch - _SCAT, nch):
                wait_scatter(g, g % _NBUF)

        def writeout(out_hbm, phase):
            pltpu.sync_copy(acc.at[row_slice], out_hbm.at[phase].at[row_slice])
            if rem:
                @pl.when(s == _TILES - 1)
                def _():
                    pltpu.sync_copy(acc.at[tail_slice],
                                    out_hbm.at[phase].at[tail_slice])

        for phase in (0, 1):
            zero_acc()
            plsc.subcore_barrier()

            @pl.when(c == 0)
            def _():
                accumulate(x0_hbm, src0_hbm, phase)

            @pl.when(c == 1)
            def _():
                accumulate(x1_hbm, src1_hbm, phase)

            plsc.subcore_barrier()

            @pl.when(c == 0)
            def _():
                writeout(out0_hbm, phase)

            @pl.when(c == 1)
            def _():
                writeout(out1_hbm, phase)

            if phase == 0:
                plsc.subcore_barrier()

    return seg2


def _segment_sum2(x0, x1, idx0, idx1):
    n, d = x0.shape
    dh = d // 2
    e = idx0.shape[1]
    shp = (_TILES, e // (_TILES * _CHUNK), _CHUNK)

    def prep(idx):
        src2 = (idx[0] * 2).reshape(shp)
        return jnp.stack([src2, src2 + 1]), idx[1].reshape(shp)

    src0, dst0 = prep(idx0)
    src1, dst1 = prep(idx1)
    zrows = jnp.zeros(((n // _TILES) // 8 * 8, dh), jnp.float32)
    return _make_segment_sum2(n, e, dh)(
        x0.reshape(2 * n, dh), x1.reshape(2 * n, dh),
        src0, dst0, src1, dst1, zrows)


_BLK = 1000


def _mlp1_body(x_ref, agg_ref, w1a_ref, b1a_ref, w1b_ref, b1b_ref, w2a_ref,
               t_ref):
    agg = jnp.concatenate([agg_ref[0], agg_ref[1]], axis=1)
    xa = x_ref[...] + agg
    g = jnp.maximum(
        jnp.dot(xa, w1a_ref[...], preferred_element_type=jnp.float32)
        + b1a_ref[...], 0.0)
    h = jnp.maximum(
        jnp.dot(g, w1b_ref[...], preferred_element_type=jnp.float32)
        + b1b_ref[...], 0.0)
    t_ref[...] = jnp.dot(h, w2a_ref[...], preferred_element_type=jnp.float32)


def _mlp1(x, agg_halves, p):
    n, d_in = x.shape
    hid = p["W1a"].shape[1]
    lat = p["W2a"].shape[1]
    grid = (n // _BLK,)
    full = lambda shape: pl.BlockSpec(shape, lambda i: (0,) * len(shape))
    return pl.pallas_call(
        _mlp1_body,
        grid=grid,
        in_specs=[
            pl.BlockSpec((_BLK, d_in), lambda i: (i, 0)),
            pl.BlockSpec((2, _BLK, d_in // 2), lambda i: (0, i, 0)),
            full((d_in, hid)), full((1, hid)),
            full((hid, hid)), full((1, hid)),
            full((hid, lat)),
        ],
        out_specs=pl.BlockSpec((_BLK, lat), lambda i: (i, 0)),
        out_shape=jax.ShapeDtypeStruct((n, lat), jnp.float32),
    )(x, agg_halves, p["W1a"], p["b1a"].reshape(1, -1), p["W1b"],
      p["b1b"].reshape(1, -1), p["W2a"])


def _mlp2_body(t_ref, aggt_ref, b2a_ref, w2b_ref, b2b_ref, wp_ref, bp_ref,
               o_ref):
    aggt = jnp.concatenate([aggt_ref[0], aggt_ref[1]], axis=1)
    z = jnp.maximum(t_ref[...] + aggt + b2a_ref[...], 0.0)
    h2 = jnp.dot(z, w2b_ref[...], preferred_element_type=jnp.float32) \
        + b2b_ref[...]
    o_ref[...] = jnp.dot(h2, wp_ref[...], preferred_element_type=jnp.float32) \
        + bp_ref[...]


def _mlp2(t, aggt_halves, p):
    n, lat = t.shape
    grid = (n // _BLK,)
    full = lambda shape: pl.BlockSpec(shape, lambda i: (0,) * len(shape))
    return pl.pallas_call(
        _mlp2_body,
        grid=grid,
        in_specs=[
            pl.BlockSpec((_BLK, lat), lambda i: (i, 0)),
            pl.BlockSpec((2, _BLK, lat // 2), lambda i: (0, i, 0)),
            full((1, lat)),
            full((lat, lat)), full((1, lat)),
            full((lat, lat)), full((1, lat)),
        ],
        out_specs=pl.BlockSpec((_BLK, lat), lambda i: (i, 0)),
        out_shape=jax.ShapeDtypeStruct((n, lat), jnp.float32),
    )(t, aggt_halves, p["b2a"].reshape(1, -1), p["W2b"],
      p["b2b"].reshape(1, -1), p["Wp"], p["bp"].reshape(1, -1))


def kernel(emg_x, eeg_x, emg_edge_index, eeg_edge_index, emg_params,
           eeg_params):
    agg_emg, agg_eeg = _segment_sum2(
        emg_x, eeg_x, emg_edge_index, eeg_edge_index)
    t_emg = _mlp1(emg_x, agg_emg, emg_params)
    t_eeg = _mlp1(eeg_x, agg_eeg, eeg_params)
    aggt_emg, aggt_eeg = _segment_sum2(
        t_emg, t_eeg, emg_edge_index, eeg_edge_index)
    o_emg = _mlp2(t_emg, aggt_emg, emg_params)
    o_eeg = _mlp2(t_eeg, aggt_eeg, eeg_params)
    return jnp.concatenate([o_emg, o_eeg], axis=0)
